# baseline (speedup 1.0000x reference)
import functools
import numpy as np
import jax
import jax.numpy as jnp
from jax.experimental import pallas as pl


N = 784
K = 25
NCTX = 1024
B = 32
NBLK = 7
BLKN = 112
BAND = 392
SUB = 14
_CHANS = [(4, 16), (16, 32), (32, 16), (16, 2)]

_BSTART = [28 * min(max(4 * b - 5, 0), 14) for b in range(NBLK)]

_DIAG = np.zeros((128, 200), np.float32)
for _m in range(16):
    for _a in range(8):
        _DIAG[_m * 8 + _a, _a * 25:(_a + 1) * 25] = 1.0


def _build_statics():
    i = jnp.linspace(-14.0, 14.0, 28)
    gx, gy = jnp.meshgrid(i, i, indexing='ij')
    grid = jnp.stack([gx, gy], axis=-1).astype(jnp.float32).reshape(-1, 2)
    d2 = jnp.sum((grid[:, None, :] - grid[None, :, :]) ** 2, axis=-1)
    _, nbhd_idx = jax.lax.top_k(-d2, 25)
    ab = grid[nbhd_idx] - grid[:, None, :]
    return grid, nbhd_idx.astype(jnp.int32), ab


def _swish(x):
    return x * jax.nn.sigmoid(x)


def _softplus(x):
    return jnp.maximum(x, 0.0) + jnp.log1p(jnp.exp(-jnp.abs(x)))


def _dotT(a, b):
    return jax.lax.dot_general(a, b, (((0,), (0,)), ((), ())),
                               preferred_element_type=jnp.float32, precision=jax.lax.Precision.HIGHEST)


def _dotL(a, b):
    return jax.lax.dot_general(a, b, (((1,), (1,)), ((), ())),
                               preferred_element_type=jnp.float32, precision=jax.lax.Precision.HIGHEST)


def _prep_body(ab_ref, *refs):
    abT = ab_ref[...]
    for li in range(4):
        W1T, b1, W2T, b2, W3T, b3 = refs[li * 6:li * 6 + 6]
        w1 = W1T[...]
        h = _swish(w1[:, 0:1] * abT[0:1, :] + w1[:, 1:2] * abT[1:2, :]
                   + b1[...])
        h = _swish(jnp.dot(W2T[...], h,
                           preferred_element_type=jnp.float32, precision=jax.lax.Precision.HIGHEST) + b2[...])
        h = _swish(jnp.dot(W3T[...], h,
                           preferred_element_type=jnp.float32, precision=jax.lax.Precision.HIGHEST) + b3[...])
        refs[24 + li][...] = h


def _run_prep(params, ab):
    ins = [ab.reshape(N * K, 2).T]
    for li in range(4):
        l = 'l%d' % li
        ins += [params[l + '_wnW1'].T, params[l + '_wnb1'].reshape(32, 1),
                params[l + '_wnW2'].T, params[l + '_wnb2'].reshape(32, 1),
                params[l + '_wnW3'].T, params[l + '_wnb3'].reshape(16, 1)]
    out_shape = [jax.ShapeDtypeStruct((16, N * K), jnp.float32)] * 4
    return pl.pallas_call(_prep_body, out_shape=out_shape)(*ins)


def _prep_band_body(w_ref, idx_ref, dg_ref, a_ref):
    blk = pl.program_id(1)
    start = jnp.int32(0)
    for b in range(NBLK):
        start = jnp.where(blk == b, jnp.int32(_BSTART[b]), start)
    dg = dg_ref[...]
    lane = jax.lax.broadcasted_iota(jnp.int32, (200, BAND), 1)
    for s in range(SUB):
        wsub = w_ref[0, 0][:, s * 200:(s + 1) * 200]
        wrep = jnp.broadcast_to(wsub[:, None, :], (16, 8, 200)).reshape(128, 200)
        wd = wrep * dg
        idx = idx_ref[0, s * 200:(s + 1) * 200, :]
        S = jnp.where(lane == (idx - start), 1.0, 0.0)
        asub = jnp.dot(wd, S, preferred_element_type=jnp.float32, precision=jax.lax.Precision.HIGHEST)
        for m in range(16):
            a_ref[0, 0, m * BLKN + s * 8:m * BLKN + s * 8 + 8, :] = \
                asub[m * 8:(m + 1) * 8, :]


def _run_prep_band(wT4, nbhd):
    wT4 = wT4.reshape(4, 16, NBLK, BLKN * K).transpose(0, 2, 1, 3)
    idx7 = nbhd.reshape(NBLK, BLKN * K, 1)
    return pl.pallas_call(
        _prep_band_body,
        grid=(4, NBLK),
        in_specs=[
            pl.BlockSpec((1, 1, 16, BLKN * K), lambda l, b: (l, b, 0, 0)),
            pl.BlockSpec((1, BLKN * K, 1), lambda l, b: (b, 0, 0)),
            pl.BlockSpec((128, 200), lambda l, b: (0, 0)),
        ],
        out_specs=pl.BlockSpec((1, 1, 16 * BLKN, BAND),
                               lambda l, b: (l, b, 0, 0)),
        out_shape=jax.ShapeDtypeStruct((4, NBLK, 16 * BLKN, BAND),
                                       jnp.float32),
    )(wT4, idx7, jnp.asarray(_DIAG))


def _encoder_body(cc_ref, cv_ref, g_ref, sc_ref, out_ref):
    g = g_ref[...]
    ccT = cc_ref[0]
    cv = cv_ref[0]
    ls2 = sc_ref[0:1, 0:1] * sc_ref[0:1, 0:1]
    os_ = sc_ref[0:1, 1:2]
    d2 = ((g[:, 0:1] - ccT[0:1, :]) ** 2
          + (g[:, 1:2] - ccT[1:2, :]) ** 2)
    psi = os_ * jnp.exp(-0.5 * d2 / ls2)
    h0 = jnp.sum(psi, axis=1, keepdims=True)
    h1 = jnp.dot(psi, cv, preferred_element_type=jnp.float32, precision=jax.lax.Precision.HIGHEST)
    out_ref[0] = jnp.concatenate([g, h0, h1 / (h0 + 1e-8)], axis=1)


def _run_encoder(ctx_coords, ctx_values, grid, scalars):
    return pl.pallas_call(
        _encoder_body,
        grid=(B,),
        in_specs=[
            pl.BlockSpec((1, 2, NCTX), lambda b: (b, 0, 0)),
            pl.BlockSpec((1, NCTX, 1), lambda b: (b, 0, 0)),
            pl.BlockSpec((N, 2), lambda b: (0, 0)),
            pl.BlockSpec((1, 4), lambda b: (0, 0)),
        ],
        out_specs=pl.BlockSpec((1, N, 4), lambda b: (b, 0, 0)),
        out_shape=jax.ShapeDtypeStruct((B, N, 4), jnp.float32),
    )(jnp.transpose(ctx_coords, (0, 2, 1)), ctx_values, grid, scalars)


def _conv_body(cin, co, last, f_ref, a_ref, lw_ref, lb_ref, out_ref):
    v = f_ref[0]
    lw = lw_ref[...]
    lb = lb_ref[...]
    for blk in range(NBLK):
        vb = v[_BSTART[blk]:_BSTART[blk] + BAND, :]
        p = jnp.dot(a_ref[blk].astype(jnp.bfloat16), vb.astype(jnp.bfloat16),
                    preferred_element_type=jnp.float32)
        p3 = p.reshape(16, BLKN, cin)
        flat = jnp.concatenate([p3[m] for m in range(16)],
                               axis=1)
        out = jnp.dot(flat.astype(jnp.bfloat16), lw.astype(jnp.bfloat16),
                      preferred_element_type=jnp.float32) + lb
        if not last:
            out = _swish(out)
        out_ref[0, blk * BLKN:(blk + 1) * BLKN, :] = out


def _run_conv(f, a_l, lw, lb, cin, co, last):
    body = functools.partial(_conv_body, cin, co, last)
    return pl.pallas_call(
        body,
        grid=(B,),
        in_specs=[
            pl.BlockSpec((1, N, cin), lambda b: (b, 0, 0)),
            pl.BlockSpec((NBLK, 16 * BLKN, BAND), lambda b: (0, 0, 0)),
            pl.BlockSpec((16 * cin, co), lambda b: (0, 0)),
            pl.BlockSpec((1, co), lambda b: (0, 0)),
        ],
        out_specs=pl.BlockSpec((1, N, co), lambda b: (b, 0, 0)),
        out_shape=jax.ShapeDtypeStruct((B, N, co), jnp.float32),
    )(f, a_l, lw, lb)


def _decoder_body(tc_ref, ft_ref, g_ref, sc_ref, mu_ref, sm_ref):
    g = g_ref[...]
    tcT = tc_ref[0]
    fT = ft_ref[0]
    ls2 = sc_ref[0:1, 2:3] * sc_ref[0:1, 2:3]
    os_ = sc_ref[0:1, 3:4]
    d2T = ((g[:, 0:1] - tcT[0:1, :]) ** 2
           + (g[:, 1:2] - tcT[1:2, :]) ** 2)
    k_rhoT = os_ * jnp.exp(-0.5 * d2T / ls2)
    f_muT = fT[0:1, :]
    f_spT = _softplus(fT[1:2, :])
    muT = jnp.dot(f_muT, k_rhoT, preferred_element_type=jnp.float32, precision=jax.lax.Precision.HIGHEST)
    sigT = jnp.dot(f_spT, k_rhoT, preferred_element_type=jnp.float32, precision=jax.lax.Precision.HIGHEST)
    mu_ref[0] = muT
    r = jax.lax.broadcasted_iota(jnp.int32, (N, N), 0)
    c = jax.lax.broadcasted_iota(jnp.int32, (N, N), 1)
    sm_ref[0] = jnp.where(r == c, jnp.broadcast_to(sigT, (N, N)), 0.0)


def _run_decoder(tgt_coords, f, grid, scalars):
    return pl.pallas_call(
        _decoder_body,
        grid=(B,),
        in_specs=[
            pl.BlockSpec((1, 2, N), lambda b: (b, 0, 0)),
            pl.BlockSpec((1, 2, N), lambda b: (b, 0, 0)),
            pl.BlockSpec((N, 2), lambda b: (0, 0)),
            pl.BlockSpec((1, 4), lambda b: (0, 0)),
        ],
        out_specs=[
            pl.BlockSpec((1, 1, N), lambda b: (b, 0, 0)),
            pl.BlockSpec((1, N, N), lambda b: (b, 0, 0)),
        ],
        out_shape=[
            jax.ShapeDtypeStruct((B, 1, N), jnp.float32),
            jax.ShapeDtypeStruct((B, N, N), jnp.float32),
        ],
    )(jnp.transpose(tgt_coords, (0, 2, 1)), jnp.transpose(f, (0, 2, 1)),
      grid, scalars)


def _weightnet(ab, p, l):
    h = _swish(ab @ p[l + '_wnW1'] + p[l + '_wnb1'])
    h = _swish(h @ p[l + '_wnW2'] + p[l + '_wnb2'])
    h = _swish(h @ p[l + '_wnW3'] + p[l + '_wnb3'])
    return h


def kernel(ctx_coords, ctx_values, ctx_mask, tgt_coords, params):
    grid, nbhd, ab = _build_statics()
    ab2 = ab.reshape(N * K, 2)
    wT = [_weightnet(ab2, params, 'l%d' % li).T for li in range(4)]
    a_band = _run_prep_band(jnp.stack(wT), nbhd)
    scalars = jnp.stack([params['psi_ls'], params['psi_os'],
                         params['rho_ls'], params['rho_os']]).reshape(1, 4)
    f = _run_encoder(ctx_coords, ctx_values, grid, scalars)
    for li, (cin, co) in enumerate(_CHANS):
        l = 'l%d' % li
        f = _run_conv(f, a_band[li], params[l + '_linW'],
                      params[l + '_linb'].reshape(1, co),
                      cin, co, last=(li == 3))
    mu, sm = _run_decoder(tgt_coords, f, grid, scalars)
    return (mu.reshape(B, N), sm)


# speedup vs baseline: 1.0345x; 1.0345x over previous
import functools
import numpy as np
import jax
import jax.numpy as jnp
from jax.experimental import pallas as pl


N = 784
K = 25
NCTX = 1024
B = 32
NBLK = 7
BLKN = 112
BAND = 392
SUB = 14
_CHANS = [(4, 16), (16, 32), (32, 16), (16, 2)]

_BSTART = [28 * min(max(4 * b - 5, 0), 14) for b in range(NBLK)]

_DIAG = np.zeros((128, 200), np.float32)
for _m in range(16):
    for _a in range(8):
        _DIAG[_m * 8 + _a, _a * 25:(_a + 1) * 25] = 1.0


def _build_statics():
    i = jnp.linspace(-14.0, 14.0, 28)
    gx, gy = jnp.meshgrid(i, i, indexing='ij')
    grid = jnp.stack([gx, gy], axis=-1).astype(jnp.float32).reshape(-1, 2)
    d2 = jnp.sum((grid[:, None, :] - grid[None, :, :]) ** 2, axis=-1)
    _, nbhd_idx = jax.lax.top_k(-d2, 25)
    ab = grid[nbhd_idx] - grid[:, None, :]
    return grid, nbhd_idx.astype(jnp.int32), ab


def _swish(x):
    return x * jax.nn.sigmoid(x)


def _softplus(x):
    return jnp.maximum(x, 0.0) + jnp.log1p(jnp.exp(-jnp.abs(x)))


def _dotT(a, b):
    return jax.lax.dot_general(a, b, (((0,), (0,)), ((), ())),
                               preferred_element_type=jnp.float32, precision=jax.lax.Precision.HIGHEST)


def _dotL(a, b):
    return jax.lax.dot_general(a, b, (((1,), (1,)), ((), ())),
                               preferred_element_type=jnp.float32, precision=jax.lax.Precision.HIGHEST)


def _prep_body(ab_ref, *refs):
    abT = ab_ref[...]
    for li in range(4):
        W1T, b1, W2T, b2, W3T, b3 = refs[li * 6:li * 6 + 6]
        w1 = W1T[...]
        h = _swish(w1[:, 0:1] * abT[0:1, :] + w1[:, 1:2] * abT[1:2, :]
                   + b1[...])
        h = _swish(jnp.dot(W2T[...], h,
                           preferred_element_type=jnp.float32, precision=jax.lax.Precision.HIGHEST) + b2[...])
        h = _swish(jnp.dot(W3T[...], h,
                           preferred_element_type=jnp.float32, precision=jax.lax.Precision.HIGHEST) + b3[...])
        refs[24 + li][...] = h


def _run_prep(params, ab):
    ins = [ab.reshape(N * K, 2).T]
    for li in range(4):
        l = 'l%d' % li
        ins += [params[l + '_wnW1'].T, params[l + '_wnb1'].reshape(32, 1),
                params[l + '_wnW2'].T, params[l + '_wnb2'].reshape(32, 1),
                params[l + '_wnW3'].T, params[l + '_wnb3'].reshape(16, 1)]
    out_shape = [jax.ShapeDtypeStruct((16, N * K), jnp.float32)] * 4
    return pl.pallas_call(_prep_body, out_shape=out_shape)(*ins)


def _prep_band_body(w_ref, idx_ref, dg_ref, a_ref):
    blk = pl.program_id(1)
    start = jnp.int32(0)
    for b in range(NBLK):
        start = jnp.where(blk == b, jnp.int32(_BSTART[b]), start)
    dg = dg_ref[...]
    lane = jax.lax.broadcasted_iota(jnp.int32, (200, BAND), 1)
    for s in range(SUB):
        wsub = w_ref[0, 0][:, s * 200:(s + 1) * 200]
        wrep = jnp.broadcast_to(wsub[:, None, :], (16, 8, 200)).reshape(128, 200)
        wd = wrep * dg
        idx = idx_ref[0, s * 200:(s + 1) * 200, :]
        S = jnp.where(lane == (idx - start), 1.0, 0.0)
        asub = jnp.dot(wd, S, preferred_element_type=jnp.float32,
                       precision=jax.lax.Precision.HIGHEST)
        asub = asub.astype(jnp.bfloat16)
        for m in range(16):
            a_ref[0, 0, m * BLKN + s * 8:m * BLKN + s * 8 + 8, :] = \
                asub[m * 8:(m + 1) * 8, :]


def _run_prep_band(wT4, nbhd):
    wT4 = wT4.reshape(4, 16, NBLK, BLKN * K).transpose(0, 2, 1, 3)
    idx7 = nbhd.reshape(NBLK, BLKN * K, 1)
    return pl.pallas_call(
        _prep_band_body,
        grid=(4, NBLK),
        in_specs=[
            pl.BlockSpec((1, 1, 16, BLKN * K), lambda l, b: (l, b, 0, 0)),
            pl.BlockSpec((1, BLKN * K, 1), lambda l, b: (b, 0, 0)),
            pl.BlockSpec((128, 200), lambda l, b: (0, 0)),
        ],
        out_specs=pl.BlockSpec((1, 1, 16 * BLKN, BAND),
                               lambda l, b: (l, b, 0, 0)),
        out_shape=jax.ShapeDtypeStruct((4, NBLK, 16 * BLKN, BAND),
                                       jnp.bfloat16),
    )(wT4, idx7, jnp.asarray(_DIAG))


def _encoder_body(cc_ref, cv_ref, g_ref, sc_ref, out_ref):
    g = g_ref[...]
    ccT = cc_ref[0]
    cv = cv_ref[0]
    ls2 = sc_ref[0:1, 0:1] * sc_ref[0:1, 0:1]
    os_ = sc_ref[0:1, 1:2]
    d2 = ((g[:, 0:1] - ccT[0:1, :]) ** 2
          + (g[:, 1:2] - ccT[1:2, :]) ** 2)
    psi = os_ * jnp.exp(-0.5 * d2 / ls2)
    h0 = jnp.sum(psi, axis=1, keepdims=True)
    h1 = jnp.dot(psi, cv, preferred_element_type=jnp.float32, precision=jax.lax.Precision.HIGHEST)
    out_ref[0] = jnp.concatenate([g, h0, h1 / (h0 + 1e-8)], axis=1)


def _run_encoder(ctx_coords, ctx_values, grid, scalars):
    return pl.pallas_call(
        _encoder_body,
        grid=(B,),
        in_specs=[
            pl.BlockSpec((1, 2, NCTX), lambda b: (b, 0, 0)),
            pl.BlockSpec((1, NCTX, 1), lambda b: (b, 0, 0)),
            pl.BlockSpec((N, 2), lambda b: (0, 0)),
            pl.BlockSpec((1, 4), lambda b: (0, 0)),
        ],
        out_specs=pl.BlockSpec((1, N, 4), lambda b: (b, 0, 0)),
        out_shape=jax.ShapeDtypeStruct((B, N, 4), jnp.float32),
    )(jnp.transpose(ctx_coords, (0, 2, 1)), ctx_values, grid, scalars)


def _conv_body(cin, co, last, f_ref, a_ref, lw_ref, lb_ref, out_ref):
    v = f_ref[0]
    lw = lw_ref[...]
    lb = lb_ref[...]
    for blk in range(NBLK):
        vb = v[_BSTART[blk]:_BSTART[blk] + BAND, :]
        p = jnp.dot(a_ref[blk], vb.astype(jnp.bfloat16),
                    preferred_element_type=jnp.float32)
        p3 = p.reshape(16, BLKN, cin)
        flat = jnp.concatenate([p3[m] for m in range(16)],
                               axis=1)
        out = jnp.dot(flat.astype(jnp.bfloat16), lw.astype(jnp.bfloat16),
                      preferred_element_type=jnp.float32) + lb
        if not last:
            out = _swish(out)
        out_ref[0, blk * BLKN:(blk + 1) * BLKN, :] = out


def _run_conv(f, a_l, lw, lb, cin, co, last):
    body = functools.partial(_conv_body, cin, co, last)
    return pl.pallas_call(
        body,
        grid=(B,),
        in_specs=[
            pl.BlockSpec((1, N, cin), lambda b: (b, 0, 0)),
            pl.BlockSpec((NBLK, 16 * BLKN, BAND), lambda b: (0, 0, 0)),
            pl.BlockSpec((16 * cin, co), lambda b: (0, 0)),
            pl.BlockSpec((1, co), lambda b: (0, 0)),
        ],
        out_specs=pl.BlockSpec((1, N, co), lambda b: (b, 0, 0)),
        out_shape=jax.ShapeDtypeStruct((B, N, co), jnp.float32),
    )(f, a_l, lw, lb)


def _decoder_body(tc_ref, ft_ref, g_ref, sc_ref, mu_ref, sm_ref):
    g = g_ref[...]
    tcT = tc_ref[0]
    fT = ft_ref[0]
    ls2 = sc_ref[0:1, 2:3] * sc_ref[0:1, 2:3]
    os_ = sc_ref[0:1, 3:4]
    d2T = ((g[:, 0:1] - tcT[0:1, :]) ** 2
           + (g[:, 1:2] - tcT[1:2, :]) ** 2)
    k_rhoT = os_ * jnp.exp(-0.5 * d2T / ls2)
    f_muT = fT[0:1, :]
    f_spT = _softplus(fT[1:2, :])
    muT = jnp.dot(f_muT, k_rhoT, preferred_element_type=jnp.float32, precision=jax.lax.Precision.HIGHEST)
    sigT = jnp.dot(f_spT, k_rhoT, preferred_element_type=jnp.float32, precision=jax.lax.Precision.HIGHEST)
    mu_ref[0] = muT
    r = jax.lax.broadcasted_iota(jnp.int32, (N, N), 0)
    c = jax.lax.broadcasted_iota(jnp.int32, (N, N), 1)
    sm_ref[0] = jnp.where(r == c, jnp.broadcast_to(sigT, (N, N)), 0.0)


def _run_decoder(tgt_coords, f, grid, scalars):
    return pl.pallas_call(
        _decoder_body,
        grid=(B,),
        in_specs=[
            pl.BlockSpec((1, 2, N), lambda b: (b, 0, 0)),
            pl.BlockSpec((1, 2, N), lambda b: (b, 0, 0)),
            pl.BlockSpec((N, 2), lambda b: (0, 0)),
            pl.BlockSpec((1, 4), lambda b: (0, 0)),
        ],
        out_specs=[
            pl.BlockSpec((1, 1, N), lambda b: (b, 0, 0)),
            pl.BlockSpec((1, N, N), lambda b: (b, 0, 0)),
        ],
        out_shape=[
            jax.ShapeDtypeStruct((B, 1, N), jnp.float32),
            jax.ShapeDtypeStruct((B, N, N), jnp.float32),
        ],
    )(jnp.transpose(tgt_coords, (0, 2, 1)), jnp.transpose(f, (0, 2, 1)),
      grid, scalars)


def _weightnet(ab, p, l):
    h = _swish(ab @ p[l + '_wnW1'] + p[l + '_wnb1'])
    h = _swish(h @ p[l + '_wnW2'] + p[l + '_wnb2'])
    h = _swish(h @ p[l + '_wnW3'] + p[l + '_wnb3'])
    return h


def kernel(ctx_coords, ctx_values, ctx_mask, tgt_coords, params):
    grid, nbhd, ab = _build_statics()
    ab2 = ab.reshape(N * K, 2)
    wT = [_weightnet(ab2, params, 'l%d' % li).T for li in range(4)]
    a_band = _run_prep_band(jnp.stack(wT), nbhd)
    scalars = jnp.stack([params['psi_ls'], params['psi_os'],
                         params['rho_ls'], params['rho_os']]).reshape(1, 4)
    f = _run_encoder(ctx_coords, ctx_values, grid, scalars)
    for li, (cin, co) in enumerate(_CHANS):
        l = 'l%d' % li
        f = _run_conv(f, a_band[li], params[l + '_linW'],
                      params[l + '_linb'].reshape(1, co),
                      cin, co, last=(li == 3))
    mu, sm = _run_decoder(tgt_coords, f, grid, scalars)
    return (mu.reshape(B, N), sm)
